# baseline (speedup 1.0000x reference)
import functools

import jax
import jax.numpy as jnp
from jax import lax
from jax.experimental import pallas as pl
from jax.experimental.pallas import tpu as pltpu
from jax.experimental.pallas import tpu_sc as plsc

N = 10000
E = 640000
G = 64

NC = 2
NS = 16
NW = NC * NS

NPAD = 10240
EPAD = 655360
EP_ROWS = EPAD // 128
BURST = 8
RPT = NPAD // NS
R0 = 160
R1 = 160

_f32 = jnp.float32


def _make_agg(d, with_deg, dt=jnp.float32):
    out_type = [jax.ShapeDtypeStruct((NC, NPAD, d), dt)]
    scratch = [
        pltpu.VMEM((BURST, 128), jnp.int32),
        pltpu.VMEM((BURST, 128), jnp.int32),
        pltpu.VMEM((BURST, 128), jnp.int32),
        pltpu.VMEM((BURST, 128), jnp.int32),
        pltpu.VMEM((BURST * 128, d), dt),
        pltpu.VMEM((BURST * 128, d), dt),
        pltpu.VMEM_SHARED((NPAD, d), dt),
        pltpu.SemaphoreType.DMA,
        pltpu.SemaphoreType.DMA,
        pltpu.SemaphoreType.DMA,
        pltpu.SemaphoreType.DMA,
    ]
    if with_deg:
        out_type.append(jax.ShapeDtypeStruct((NC, NPAD, 8), _f32))
        scratch += [
            pltpu.VMEM((128, 8), _f32),
            pltpu.VMEM_SHARED((NPAD, 8), _f32),
        ]

    def body(*refs):
        if with_deg:
            (feat, srcp, dstp, zrs, zrsd, ones, out, degout,
             src0, dst0, src1, dst1, rows0, rows1, acc,
             sem_g0, sem_g1, sem_s0, sem_s1, ones_v, dacc) = refs
        else:
            (feat, srcp, dstp, zrs, out,
             src0, dst0, src1, dst1, rows0, rows1, acc,
             sem_g0, sem_g1, sem_s0, sem_s1) = refs
            ones_v = dacc = None

        c = lax.axis_index("c")
        s = lax.axis_index("s")

        pltpu.sync_copy(zrs.at[pl.ds(s * RPT, RPT), pl.ds(0, d)],
                        acc.at[pl.ds(s * RPT, RPT)])
        if with_deg:
            pltpu.sync_copy(zrsd.at[pl.ds(s * RPT, RPT)],
                            dacc.at[pl.ds(s * RPT, RPT)])
            pltpu.sync_copy(ones, ones_v)
        plsc.subcore_barrier()

        rw = jnp.where(c == 0, R0, R1)
        base = jnp.where(c == 0, s * R0, NS * R0 + s * R1)
        pairs = rw // (2 * BURST)

        def load_idx(r, sbuf, dbuf):
            pltpu.sync_copy(srcp.at[pl.ds(r, BURST)], sbuf)
            pltpu.sync_copy(dstp.at[pl.ds(r, BURST)], dbuf)

        def fire_g(sbuf, rbuf, sem):
            for j in range(BURST):
                pltpu.async_copy(feat.at[sbuf.at[j]],
                                 rbuf.at[pl.ds(j * 128, 128)], sem)

        def wait_g(sbuf, rbuf, sem):
            for j in range(BURST):
                pltpu.make_async_copy(feat.at[sbuf.at[j]],
                                      rbuf.at[pl.ds(j * 128, 128)],
                                      sem).wait()

        def fire_s(rbuf, dbuf, sem):
            for j in range(BURST):
                pltpu.async_copy(rbuf.at[pl.ds(j * 128, 128)],
                                 acc.at[dbuf.at[j]], sem, add=True)
            if with_deg:
                for j in range(BURST):
                    pltpu.async_copy(ones_v, dacc.at[dbuf.at[j]], sem,
                                     add=True)

        def wait_s(rbuf, dbuf, sem):
            for j in range(BURST):
                pltpu.make_async_copy(rbuf.at[pl.ds(j * 128, 128)],
                                      acc.at[dbuf.at[j]], sem).wait()
            if with_deg:
                for j in range(BURST):
                    pltpu.make_async_copy(ones_v, dacc.at[dbuf.at[j]],
                                          sem).wait()

        load_idx(base, src0, dst0)
        fire_g(src0, rows0, sem_g0)
        load_idx(base + BURST, src1, dst1)
        fire_g(src1, rows1, sem_g1)

        outer_n = rw // BURST

        def pair(t, carry):
            r0 = base + lax.rem(2 * t + 2, outer_n) * BURST
            r1 = base + lax.rem(2 * t + 3, outer_n) * BURST
            wait_g(src0, rows0, sem_g0)
            fire_s(rows0, dst0, sem_s0)
            wait_g(src1, rows1, sem_g1)
            fire_s(rows1, dst1, sem_s1)
            wait_s(rows0, dst0, sem_s0)
            load_idx(r0, src0, dst0)
            fire_g(src0, rows0, sem_g0)
            wait_s(rows1, dst1, sem_s1)
            load_idx(r1, src1, dst1)
            fire_g(src1, rows1, sem_g1)
            return carry

        lax.fori_loop(0, pairs, pair, 0)
        wait_g(src0, rows0, sem_g0)
        wait_g(src1, rows1, sem_g1)
        plsc.subcore_barrier()

        pltpu.sync_copy(acc.at[pl.ds(s * RPT, RPT)],
                        out.at[c, pl.ds(s * RPT, RPT)])
        if with_deg:
            pltpu.sync_copy(dacc.at[pl.ds(s * RPT, RPT)],
                            degout.at[c, pl.ds(s * RPT, RPT)])

    def run(*args):
        mesh = plsc.VectorSubcoreMesh(
            core_axis_name="c", subcore_axis_name="s",
            num_cores=NC, num_subcores=NS)
        return pl.kernel(body, out_type=out_type, mesh=mesh,
                         scratch_types=scratch,
                         compiler_params=pltpu.CompilerParams(
                             use_tc_tiling_on_sc=False))(*args)

    return run


_agg16_deg = _make_agg(16, True, jnp.bfloat16)
_agg16 = _make_agg(16, False, jnp.bfloat16)
_agg32 = _make_agg(32, False, jnp.bfloat16)


def _dot(a, b):
    return jnp.dot(a, b, preferred_element_type=_f32)


def _bn_elu(z, g, be):
    mu = jnp.mean(z, axis=0, keepdims=True)
    var = jnp.mean((z - mu) * (z - mu), axis=0, keepdims=True)
    zb = g * (z - mu) * lax.rsqrt(var + 1e-5) + be
    return jnp.where(zb > 0, zb, jnp.exp(jnp.minimum(zb, 0.0)) - 1.0)


def _pre_body(x, wr, ws, xr_o, xs_o):
    xv = x[...]
    xr_o[...] = _dot(xv, wr[...]).astype(jnp.bfloat16)
    xs_o[...] = _dot(xv, ws[...])


def _layer1_body(m0, m1, d0, d1, xs, b, g, be, h_o, hb_o, invd_o):
    deg = d0[...] + d1[...]
    invd = 1.0 / jnp.maximum(deg, 1.0)
    z = (m0[...].astype(_f32) + m1[...].astype(_f32)) * invd + b[...] + xs[...]
    hv = _bn_elu(z, g[...], be[...])
    h_o[...] = hv
    hb_o[...] = hv.astype(jnp.bfloat16)
    invd_o[...] = invd


def _layer_body(m0, m1, invd, h, wr, ws, b, g, be, h_o, hb_o):
    m = (m0[...].astype(_f32) + m1[...].astype(_f32)) * invd[...]
    z = _dot(m, wr[...]) + b[...] + _dot(h[...], ws[...])
    hv = _bn_elu(z, g[...], be[...])
    h_o[...] = hv
    hb_o[...] = hv.astype(jnp.bfloat16)


def _final_body(m0, m1, invd, h, wr, ws, b, g, be, batch,
                wl1, bl1, wl2, bl2, out_o):
    m = (m0[...].astype(_f32) + m1[...].astype(_f32)) * invd[...]
    z = _dot(m, wr[...]) + b[...] + _dot(h[...], ws[...])
    h4 = _bn_elu(z, g[...], be[...])
    bid = lax.broadcasted_iota(jnp.int32, (G, N), 0)
    mask = jnp.where(bid == batch[...], 1.0, 0.0)
    psum = _dot(mask, h4)
    cnt = jnp.sum(mask, axis=1, keepdims=True)
    pooled = psum / jnp.maximum(cnt, 1.0)
    zz = jnp.maximum(_dot(pooled, wl1[...]) + bl1[...], 0.0)
    zz = _dot(zz, wl2[...]) + bl2[...]
    zm = jnp.max(zz, axis=1, keepdims=True)
    zs = zz - zm
    out_o[...] = zs - jnp.log(jnp.sum(jnp.exp(zs), axis=1, keepdims=True))


def _tc_call(body, out_shapes):
    return pl.pallas_call(
        body, out_shape=out_shapes,
        compiler_params=pltpu.CompilerParams(
            dimension_semantics=()),
    )


def kernel(x, edge_index, batch,
           W1r, b1, W1s, g1, be1,
           W2r, b2, W2s, g2, be2,
           W3r, b3, W3s, g3, be3,
           W4r, b4, W4s, g4, be4,
           Wl1, bl1, Wl2, bl2):
    src = edge_index[0]
    dst = edge_index[1]
    pad = EPAD - E
    srcp = jnp.concatenate(
        [src, jnp.zeros((pad,), jnp.int32)]).reshape(EP_ROWS, 128)
    dstp = jnp.concatenate(
        [dst, jnp.full((pad,), N, jnp.int32)]).reshape(EP_ROWS, 128)
    zeros16 = jnp.zeros((NPAD, 16), jnp.bfloat16)
    zeros8 = jnp.zeros((NPAD, 8), _f32)
    zeros32 = jnp.zeros((NPAD, 32), jnp.bfloat16)
    ones128 = jnp.ones((128, 8), _f32)
    batch2d = batch.reshape(1, N)

    r1 = lambda v: v.reshape(1, -1)

    xr1, xs1 = _tc_call(_pre_body, [
        jax.ShapeDtypeStruct((N, 16), jnp.bfloat16),
        jax.ShapeDtypeStruct((N, 16), _f32)])(x, W1r, W1s)
    m1p, degp = _agg16_deg(xr1, srcp, dstp, zeros16, zeros8, ones128)
    h1, h1b, invd = _tc_call(_layer1_body, [
        jax.ShapeDtypeStruct((N, 16), _f32),
        jax.ShapeDtypeStruct((N, 16), jnp.bfloat16),
        jax.ShapeDtypeStruct((N, 1), _f32)])(
            m1p[0, :N], m1p[1, :N], degp[0, :N, 0:1], degp[1, :N, 0:1],
            xs1, r1(b1), r1(g1), r1(be1))

    (m2p,) = _agg16(h1b, srcp, dstp, zeros16)
    h2, h2b = _tc_call(_layer_body, [
        jax.ShapeDtypeStruct((N, 32), _f32),
        jax.ShapeDtypeStruct((N, 32), jnp.bfloat16)])(
            m2p[0, :N], m2p[1, :N], invd, h1, W2r, W2s,
            r1(b2), r1(g2), r1(be2))

    (m3p,) = _agg32(h2b, srcp, dstp, zeros32)
    h3, h3b = _tc_call(_layer_body, [
        jax.ShapeDtypeStruct((N, 32), _f32),
        jax.ShapeDtypeStruct((N, 32), jnp.bfloat16)])(
            m3p[0, :N], m3p[1, :N], invd, h2, W3r, W3s,
            r1(b3), r1(g3), r1(be3))

    (m4p,) = _agg32(h3b, srcp, dstp, zeros32)
    out = _tc_call(_final_body, jax.ShapeDtypeStruct((G, 10), _f32))(
        m4p[0, :N], m4p[1, :N], invd, h3, W4r, W4s,
        r1(b4), r1(g4), r1(be4), batch2d,
        Wl1, r1(bl1), Wl2, r1(bl2))
    return out


# speedup vs baseline: 1.0160x; 1.0160x over previous
import jax
import jax.numpy as jnp
from jax import lax
from jax.experimental import pallas as pl
from jax.experimental.pallas import tpu as pltpu
from jax.experimental.pallas import tpu_sc as plsc

N = 10000
E = 640000
G = 64

NC = 2
NS = 16
NW = NC * NS

NPAD = 10240
EPAD = 655360
EP_ROWS = EPAD // 128
BURST = 8
RPT = NPAD // NS
R0 = 192
R1 = 128

_f32 = jnp.float32


def _make_agg(d, with_deg, dt=jnp.float32):
    out_type = [jax.ShapeDtypeStruct((NC, NPAD, d), dt)]
    scratch = [
        pltpu.VMEM((BURST, 128), jnp.int32),
        pltpu.VMEM((BURST, 128), jnp.int32),
        pltpu.VMEM((BURST, 128), jnp.int32),
        pltpu.VMEM((BURST, 128), jnp.int32),
        pltpu.VMEM((BURST * 128, d), dt),
        pltpu.VMEM((BURST * 128, d), dt),
        pltpu.VMEM_SHARED((NPAD, d), dt),
        pltpu.SemaphoreType.DMA,
        pltpu.SemaphoreType.DMA,
        pltpu.SemaphoreType.DMA,
        pltpu.SemaphoreType.DMA,
    ]
    if with_deg:
        out_type.append(jax.ShapeDtypeStruct((NC, NPAD, 8), _f32))
        scratch += [
            pltpu.VMEM((128, 8), _f32),
            pltpu.VMEM_SHARED((NPAD, 8), _f32),
        ]

    def body(*refs):
        if with_deg:
            (feat, srcp, dstp, zrs, zrsd, ones, out, degout,
             src0, dst0, src1, dst1, rows0, rows1, acc,
             sem_g0, sem_g1, sem_s0, sem_s1, ones_v, dacc) = refs
        else:
            (feat, srcp, dstp, zrs, out,
             src0, dst0, src1, dst1, rows0, rows1, acc,
             sem_g0, sem_g1, sem_s0, sem_s1) = refs
            ones_v = dacc = None

        c = lax.axis_index("c")
        s = lax.axis_index("s")

        pltpu.sync_copy(zrs.at[pl.ds(s * RPT, RPT), pl.ds(0, d)],
                        acc.at[pl.ds(s * RPT, RPT)])
        if with_deg:
            pltpu.sync_copy(zrsd.at[pl.ds(s * RPT, RPT)],
                            dacc.at[pl.ds(s * RPT, RPT)])
            pltpu.sync_copy(ones, ones_v)
        plsc.subcore_barrier()

        rw = jnp.where(c == 0, R0, R1)
        base = jnp.where(c == 0, s * R0, NS * R0 + s * R1)
        pairs = rw // (2 * BURST)

        def load_idx(r, sbuf, dbuf):
            pltpu.sync_copy(srcp.at[pl.ds(r, BURST)], sbuf)
            pltpu.sync_copy(dstp.at[pl.ds(r, BURST)], dbuf)

        def fire_g(sbuf, rbuf, sem):
            for j in range(BURST):
                pltpu.async_copy(feat.at[sbuf.at[j]],
                                 rbuf.at[pl.ds(j * 128, 128)], sem)

        def wait_g(sbuf, rbuf, sem):
            for j in range(BURST):
                pltpu.make_async_copy(feat.at[sbuf.at[j]],
                                      rbuf.at[pl.ds(j * 128, 128)],
                                      sem).wait()

        def fire_s(rbuf, dbuf, sem):
            for j in range(BURST):
                pltpu.async_copy(rbuf.at[pl.ds(j * 128, 128)],
                                 acc.at[dbuf.at[j]], sem, add=True)
            if with_deg:
                for j in range(BURST):
                    pltpu.async_copy(ones_v, dacc.at[dbuf.at[j]], sem,
                                     add=True)

        def wait_s(rbuf, dbuf, sem):
            for j in range(BURST):
                pltpu.make_async_copy(rbuf.at[pl.ds(j * 128, 128)],
                                      acc.at[dbuf.at[j]], sem).wait()
            if with_deg:
                for j in range(BURST):
                    pltpu.make_async_copy(ones_v, dacc.at[dbuf.at[j]],
                                          sem).wait()

        load_idx(base, src0, dst0)
        fire_g(src0, rows0, sem_g0)
        load_idx(base + BURST, src1, dst1)
        fire_g(src1, rows1, sem_g1)

        outer_n = rw // BURST

        def pair(t, carry):
            r0 = base + lax.rem(2 * t + 2, outer_n) * BURST
            r1 = base + lax.rem(2 * t + 3, outer_n) * BURST
            wait_g(src0, rows0, sem_g0)
            fire_s(rows0, dst0, sem_s0)
            wait_g(src1, rows1, sem_g1)
            fire_s(rows1, dst1, sem_s1)
            wait_s(rows0, dst0, sem_s0)
            load_idx(r0, src0, dst0)
            fire_g(src0, rows0, sem_g0)
            wait_s(rows1, dst1, sem_s1)
            load_idx(r1, src1, dst1)
            fire_g(src1, rows1, sem_g1)
            return carry

        lax.fori_loop(0, pairs, pair, 0)
        wait_g(src0, rows0, sem_g0)
        wait_g(src1, rows1, sem_g1)
        plsc.subcore_barrier()

        pltpu.sync_copy(acc.at[pl.ds(s * RPT, RPT)],
                        out.at[c, pl.ds(s * RPT, RPT)])
        if with_deg:
            pltpu.sync_copy(dacc.at[pl.ds(s * RPT, RPT)],
                            degout.at[c, pl.ds(s * RPT, RPT)])

    def run(*args):
        mesh = plsc.VectorSubcoreMesh(
            core_axis_name="c", subcore_axis_name="s",
            num_cores=NC, num_subcores=NS)
        return pl.kernel(body, out_type=out_type, mesh=mesh,
                         scratch_types=scratch,
                         compiler_params=pltpu.CompilerParams(
                             use_tc_tiling_on_sc=False))(*args)

    return run


_agg16_deg = _make_agg(16, True, jnp.bfloat16)
_agg16 = _make_agg(16, False, jnp.bfloat16)
_agg32 = _make_agg(32, False, jnp.bfloat16)


def _dot(a, b):
    return jnp.dot(a, b, preferred_element_type=_f32)


def _bn_elu(z, g, be):
    mu = jnp.mean(z, axis=0, keepdims=True)
    var = jnp.mean((z - mu) * (z - mu), axis=0, keepdims=True)
    zb = g * (z - mu) * lax.rsqrt(var + 1e-5) + be
    return jnp.where(zb > 0, zb, jnp.exp(jnp.minimum(zb, 0.0)) - 1.0)


def _pre_body(x, wr, ws, xr_o, xs_o):
    xv = x[...]
    xr_o[...] = _dot(xv, wr[...]).astype(jnp.bfloat16)
    xs_o[...] = _dot(xv, ws[...])


def _layer1_body(m0, m1, d0, d1, xs, b, g, be, h_o, hb_o, invd_o):
    deg = d0[...] + d1[...]
    invd = 1.0 / jnp.maximum(deg, 1.0)
    z = (m0[...].astype(_f32) + m1[...].astype(_f32)) * invd + b[...] + xs[...]
    hv = _bn_elu(z, g[...], be[...])
    h_o[...] = hv
    hb_o[...] = hv.astype(jnp.bfloat16)
    invd_o[...] = invd


def _layer_body(m0, m1, invd, h, wr, ws, b, g, be, h_o, hb_o):
    m = (m0[...].astype(_f32) + m1[...].astype(_f32)) * invd[...]
    z = _dot(m, wr[...]) + b[...] + _dot(h[...], ws[...])
    hv = _bn_elu(z, g[...], be[...])
    h_o[...] = hv
    hb_o[...] = hv.astype(jnp.bfloat16)


def _final_body(m0, m1, invd, h, wr, ws, b, g, be, batch,
                wl1, bl1, wl2, bl2, out_o):
    m = (m0[...].astype(_f32) + m1[...].astype(_f32)) * invd[...]
    z = _dot(m, wr[...]) + b[...] + _dot(h[...], ws[...])
    h4 = _bn_elu(z, g[...], be[...])
    bid = lax.broadcasted_iota(jnp.int32, (G, N), 0)
    mask = jnp.where(bid == batch[...], 1.0, 0.0)
    psum = _dot(mask, h4)
    cnt = jnp.sum(mask, axis=1, keepdims=True)
    pooled = psum / jnp.maximum(cnt, 1.0)
    zz = jnp.maximum(_dot(pooled, wl1[...]) + bl1[...], 0.0)
    zz = _dot(zz, wl2[...]) + bl2[...]
    zm = jnp.max(zz, axis=1, keepdims=True)
    zs = zz - zm
    out_o[...] = zs - jnp.log(jnp.sum(jnp.exp(zs), axis=1, keepdims=True))


def _tc_call(body, out_shapes):
    return pl.pallas_call(
        body, out_shape=out_shapes,
        compiler_params=pltpu.CompilerParams(
            dimension_semantics=()),
    )


def kernel(x, edge_index, batch,
           W1r, b1, W1s, g1, be1,
           W2r, b2, W2s, g2, be2,
           W3r, b3, W3s, g3, be3,
           W4r, b4, W4s, g4, be4,
           Wl1, bl1, Wl2, bl2):
    src = edge_index[0]
    dst = edge_index[1]
    pad = EPAD - E
    srcp = jnp.concatenate(
        [src, jnp.zeros((pad,), jnp.int32)]).reshape(EP_ROWS, 128)
    dstp = jnp.concatenate(
        [dst, jnp.full((pad,), N, jnp.int32)]).reshape(EP_ROWS, 128)
    zeros16 = jnp.zeros((NPAD, 16), jnp.bfloat16)
    zeros8 = jnp.zeros((NPAD, 8), _f32)
    zeros32 = jnp.zeros((NPAD, 32), jnp.bfloat16)
    ones128 = jnp.ones((128, 8), _f32)
    batch2d = batch.reshape(1, N)

    r1 = lambda v: v.reshape(1, -1)

    xr1, xs1 = _tc_call(_pre_body, [
        jax.ShapeDtypeStruct((N, 16), jnp.bfloat16),
        jax.ShapeDtypeStruct((N, 16), _f32)])(x, W1r, W1s)
    m1p, degp = _agg16_deg(xr1, srcp, dstp, zeros16, zeros8, ones128)
    h1, h1b, invd = _tc_call(_layer1_body, [
        jax.ShapeDtypeStruct((N, 16), _f32),
        jax.ShapeDtypeStruct((N, 16), jnp.bfloat16),
        jax.ShapeDtypeStruct((N, 1), _f32)])(
            m1p[0, :N], m1p[1, :N], degp[0, :N, 0:1], degp[1, :N, 0:1],
            xs1, r1(b1), r1(g1), r1(be1))

    (m2p,) = _agg16(h1b, srcp, dstp, zeros16)
    h2, h2b = _tc_call(_layer_body, [
        jax.ShapeDtypeStruct((N, 32), _f32),
        jax.ShapeDtypeStruct((N, 32), jnp.bfloat16)])(
            m2p[0, :N], m2p[1, :N], invd, h1, W2r, W2s,
            r1(b2), r1(g2), r1(be2))

    (m3p,) = _agg32(h2b, srcp, dstp, zeros32)
    h3, h3b = _tc_call(_layer_body, [
        jax.ShapeDtypeStruct((N, 32), _f32),
        jax.ShapeDtypeStruct((N, 32), jnp.bfloat16)])(
            m3p[0, :N], m3p[1, :N], invd, h2, W3r, W3s,
            r1(b3), r1(g3), r1(be3))

    (m4p,) = _agg32(h3b, srcp, dstp, zeros32)
    out = _tc_call(_final_body, jax.ShapeDtypeStruct((G, 10), _f32))(
        m4p[0, :N], m4p[1, :N], invd, h3, W4r, W4s,
        r1(b4), r1(g4), r1(be4), batch2d,
        Wl1, r1(bl1), Wl2, r1(bl2))
    return out
